# baseline (speedup 1.0000x reference)
import functools

import jax
import jax.numpy as jnp
from jax.experimental import pallas as pl
from jax.experimental.pallas import tpu as pltpu

B = 256
NG = 8192
NGO = 4096
NKE = 1024
DRUG = 2048
NT = 512
H = 8

BN = 256
BK = 512


def _node_mlp(x, W1, b1, W2, b2):
    h = jnp.tanh(x[..., None] * W1 + b1)
    return jnp.sum(h * W2, axis=-1) + b2


def _edge_mean(node_vals, src, dst, num):
    msgs = node_vals[:, src]
    s = jax.ops.segment_sum(msgs.T, dst, num_segments=num)
    cnt = jax.ops.segment_sum(jnp.ones(src.shape, node_vals.dtype), dst, num_segments=num)
    return (s / jnp.maximum(cnt, 1.0)[:, None]).T


def _go_stage_body(sum_ref, cnt_ref, w_ref, b_ref, mW1_ref, mb1_ref, mW2_ref, mb2_ref,
                   out_ref, acc_ref):
    k = pl.program_id(1)
    nk = pl.num_programs(1)

    @pl.when(k == 0)
    def _():
        acc_ref[...] = jnp.zeros_like(acc_ref)

    a = sum_ref[...] / jnp.maximum(cnt_ref[...], 1.0)
    acc_ref[...] += jnp.dot(a, w_ref[...], preferred_element_type=jnp.float32)

    @pl.when(k == nk - 1)
    def _():
        go = jnp.tanh(acc_ref[...] + b_ref[...])
        acc = jnp.zeros_like(go)
        for i in range(H):
            acc += jnp.tanh(go * mW1_ref[0, i] + mb1_ref[0, i]) * mW2_ref[0, i]
        out_ref[...] = acc + mb2_ref[0, 0]


def _go_stage(go_sum, go_cnt, W, b, mW1, mb1, mW2, mb2):
    grid = (NGO // BN, NGO // BK)
    return pl.pallas_call(
        _go_stage_body,
        grid=grid,
        in_specs=[
            pl.BlockSpec((B, BK), lambda j, k: (0, k)),
            pl.BlockSpec((1, BK), lambda j, k: (0, k)),
            pl.BlockSpec((BK, BN), lambda j, k: (k, j)),
            pl.BlockSpec((1, BN), lambda j, k: (0, j)),
            pl.BlockSpec((1, H), lambda j, k: (0, 0)),
            pl.BlockSpec((1, H), lambda j, k: (0, 0)),
            pl.BlockSpec((1, H), lambda j, k: (0, 0)),
            pl.BlockSpec((1, 1), lambda j, k: (0, 0)),
        ],
        out_specs=pl.BlockSpec((B, BN), lambda j, k: (0, j)),
        out_shape=jax.ShapeDtypeStruct((B, NGO), jnp.float32),
        scratch_shapes=[pltpu.VMEM((B, BN), jnp.float32)],
    )(go_sum, go_cnt, W, b, mW1, mb1, mW2, mb2)


def kernel(input_tensor, gene_W1, gene_b1, gene_W2, gene_b2, go_enc_W, go_enc_b,
           go_dec_W1, go_dec_b1, go_dec_W2, go_dec_b2,
           ke_ws0, ke_wn0, ke_b0, ke_ws1, ke_wn1, ke_b1,
           kel_W1, kel_b1, kel_W2, kel_b2,
           bio_W1, bio_b1, bio_W2, bio_b2,
           drug_W1, drug_b1, drug_W2, drug_b2,
           pred_W, pred_b,
           gene_go, go_ke, ke_ke, tissue):
    gene = _node_mlp(input_tensor[:, :NG], gene_W1, gene_b1, gene_W2, gene_b2)

    msgs = gene[:, gene_go[0]]
    go_sum = jax.ops.segment_sum(msgs.T, gene_go[1], num_segments=NGO).T
    go_cnt = jax.ops.segment_sum(jnp.ones((gene_go.shape[1],), jnp.float32),
                                 gene_go[1], num_segments=NGO)

    go = _go_stage(go_sum, go_cnt[None, :], go_enc_W, go_enc_b[None, :],
                   go_dec_W1[None, :], go_dec_b1[None, :],
                   go_dec_W2[None, :], go_dec_b2.reshape(1, 1))

    ke = _edge_mean(go, go_ke[0], go_ke[1], NKE)
    agg = _edge_mean(ke, ke_ke[0], ke_ke[1], NKE)
    ke = jax.nn.relu(ke * ke_ws0 + agg * ke_wn0 + ke_b0)
    agg = _edge_mean(ke, ke_ke[0], ke_ke[1], NKE)
    ke = jax.nn.relu(ke * ke_ws1 + agg * ke_wn1 + ke_b1)
    ke = _node_mlp(ke, kel_W1, kel_b1, kel_W2, kel_b2)
    bio = ke[:, tissue]
    bio = jax.nn.relu(bio @ bio_W1 + bio_b1)
    bio = jax.nn.relu(bio @ bio_W2 + bio_b2)
    drug = input_tensor[:, NG:]
    drug = jax.nn.relu(drug @ drug_W1 + drug_b1)
    drug = jax.nn.relu(drug @ drug_W2 + drug_b2)
    comb = jnp.concatenate([bio, drug], axis=-1)
    return comb @ pred_W + pred_b


# speedup vs baseline: 3.5960x; 3.5960x over previous
import functools

import jax
import jax.numpy as jnp
from jax import lax
from jax.experimental import pallas as pl
from jax.experimental.pallas import tpu as pltpu
from jax.experimental.pallas import tpu_sc as plsc

B = 256
BA = 384
NG = 8192
NGO = 4096
NKE = 1024
DRUG = 2048
NT = 512
H = 8

NS = 16
NC = 2
CH = 128


def _tc_gene_body(x_ref, w1_ref, b1_ref, w2_ref, b2_ref, o_ref):
    x = x_ref[...]
    acc = jnp.zeros_like(x)
    for i in range(H):
        acc += jnp.tanh(x * w1_ref[0, i] + b1_ref[0, i]) * w2_ref[0, i]
    o_ref[:, :B] = acc + b2_ref[0, 0]
    o_ref[:, B:] = jnp.ones((x.shape[0], BA - B), jnp.float32)


def _tc_gene(xT, w1, b1, w2, b2):
    grid = (NG // 512,)
    return pl.pallas_call(
        _tc_gene_body,
        grid=grid,
        in_specs=[
            pl.BlockSpec((512, B), lambda i: (i, 0)),
            pl.BlockSpec((1, H), lambda i: (0, 0)),
            pl.BlockSpec((1, H), lambda i: (0, 0)),
            pl.BlockSpec((1, H), lambda i: (0, 0)),
            pl.BlockSpec((1, 1), lambda i: (0, 0)),
        ],
        out_specs=pl.BlockSpec((512, BA), lambda i: (i, 0)),
        out_shape=jax.ShapeDtypeStruct((NG, BA), jnp.float32),
    )(xT, w1, b1, w2, b2)


def _sc_gene_body(node_h, src_h, dst_h, zacc_h,
                  sum_o,
                  src_v, dst_v, rows_v, sem):
    c = lax.axis_index("c")
    s = lax.axis_index("s")
    wid = c * NS + s
    rpt = NGO // NS
    pltpu.sync_copy(src_h.at[wid], src_v)
    pltpu.sync_copy(dst_h.at[wid], dst_v)
    pltpu.sync_copy(zacc_h, rows_v)
    pltpu.sync_copy(rows_v, sum_o.at[c].at[pl.ds(s * rpt, CH)])
    pltpu.sync_copy(rows_v, sum_o.at[c].at[pl.ds(s * rpt + CH, CH)])
    plsc.subcore_barrier()

    def step(i, carry):
        pltpu.async_copy(node_h.at[src_v.at[i]], rows_v, sem).wait()
        pltpu.sync_copy(rows_v, sum_o.at[c].at[dst_v.at[i]], add=True)
        return carry

    lax.fori_loop(0, 16, step, 0)


def _sc_gene(geneT, src3, dst3, zacc):
    mesh = plsc.VectorSubcoreMesh(core_axis_name="c", subcore_axis_name="s",
                                  num_cores=NC, num_subcores=NS)
    f = pl.kernel(
        _sc_gene_body,
        out_type=jax.ShapeDtypeStruct((NC, NGO, BA), jnp.float32),
        mesh=mesh,
        scratch_types=[
            pltpu.VMEM((16, CH), jnp.int32),
            pltpu.VMEM((16, CH), jnp.int32),
            pltpu.VMEM((CH, BA), jnp.float32),
            pltpu.SemaphoreType.DMA,
        ],
    )
    return f(geneT, src3, dst3, zacc)


def _tc_go_body(gsum_ref, w_ref, b_ref, mw1_ref, mb1_ref,
                mw2_ref, mb2_ref, o_ref, acc_ref):
    k = pl.program_id(0)
    nk = pl.num_programs(0)

    @pl.when(k == 0)
    def _():
        acc_ref[...] = jnp.zeros_like(acc_ref)

    g = gsum_ref[0] + gsum_ref[1]
    cn = jnp.maximum(g[:, B:B + 1], 1.0)
    a = g[:, :B] / cn
    acc_ref[...] += lax.dot_general(w_ref[...], a, (((0,), (0,)), ((), ())),
                                    preferred_element_type=jnp.float32)

    @pl.when(k == nk - 1)
    def _():
        go = jnp.tanh(acc_ref[...] + b_ref[...][:, :1])
        acc = jnp.zeros_like(go)
        for i in range(H):
            acc += jnp.tanh(go * mw1_ref[0, i] + mb1_ref[0, i]) * mw2_ref[0, i]
        o_ref[:, :B] = acc + mb2_ref[0, 0]
        o_ref[:, B:] = jnp.ones((NGO, BA - B), jnp.float32)


def _tc_go(gsum, W, b16, mw1, mb1, mw2, mb2):
    BK = 512
    grid = (NGO // BK,)
    return pl.pallas_call(
        _tc_go_body,
        grid=grid,
        in_specs=[
            pl.BlockSpec((NC, BK, BA), lambda k: (0, k, 0)),
            pl.BlockSpec((BK, NGO), lambda k: (k, 0)),
            pl.BlockSpec((NGO, 16), lambda k: (0, 0)),
            pl.BlockSpec((1, H), lambda k: (0, 0)),
            pl.BlockSpec((1, H), lambda k: (0, 0)),
            pl.BlockSpec((1, H), lambda k: (0, 0)),
            pl.BlockSpec((1, 1), lambda k: (0, 0)),
        ],
        out_specs=pl.BlockSpec((NGO, BA), lambda k: (0, 0)),
        out_shape=jax.ShapeDtypeStruct((NGO, BA), jnp.float32),
        scratch_shapes=[pltpu.VMEM((NGO, B), jnp.float32)],
    )(gsum, W, b16, mw1, mb1, mw2, mb2)


def _sc_ke_body(goT_h, gks_h, gkd_h, kks_h, kkd_h, tis_h,
                par_h, zacc_h,
                bio_o, acc1_o, acc2_o, acc3_o, ke0_o, ke1_o,
                si_v, di_v, ksi_v, kdi_v, tis_v,
                rows_v, work_v, keA_v, par_v, sem):
    s = lax.axis_index("s")
    R = NKE // NS
    r0 = s * R
    NJ = B // 16

    pltpu.sync_copy(gks_h.at[s], si_v)
    pltpu.sync_copy(gkd_h.at[s], di_v)
    pltpu.sync_copy(kks_h.at[s], ksi_v)
    pltpu.sync_copy(kkd_h.at[s], kdi_v)
    pltpu.sync_copy(tis_h.at[s], tis_v)
    pltpu.sync_copy(par_h.at[pl.ds(r0, R)], par_v)
    pltpu.sync_copy(zacc_h.at[pl.ds(0, R)], work_v)
    pltpu.sync_copy(work_v, acc1_o.at[pl.ds(r0, R)])
    pltpu.sync_copy(work_v, acc2_o.at[pl.ds(r0, R)])
    pltpu.sync_copy(work_v, acc3_o.at[pl.ds(r0, R)])
    plsc.subcore_barrier()

    def stepA(i, carry):
        pltpu.async_copy(goT_h.at[si_v.at[i]], rows_v, sem).wait()
        pltpu.sync_copy(rows_v, acc1_o.at[di_v.at[i]], add=True)
        return carry

    lax.fori_loop(0, 16, stepA, 0)
    plsc.subcore_barrier()

    pltpu.sync_copy(acc1_o.at[pl.ds(r0, R)], work_v)

    def row_div(r, carry):
        cv = jnp.maximum(work_v[r, pl.ds(B, 16)], 1.0)

        def chunk(j, c2):
            keA_v[r, pl.ds(j * 16, 16)] = work_v[r, pl.ds(j * 16, 16)] / cv
            return c2

        lax.fori_loop(0, NJ, chunk, 0)
        for j in range(NJ, BA // 16):
            keA_v[r, pl.ds(j * 16, 16)] = jnp.ones((16,), jnp.float32)
        return carry

    lax.fori_loop(0, R, row_div, 0)
    pltpu.sync_copy(keA_v, ke0_o.at[pl.ds(r0, R)])
    plsc.subcore_barrier()

    def stepB(i, carry):
        pltpu.async_copy(ke0_o.at[ksi_v.at[i]], rows_v, sem).wait()
        pltpu.sync_copy(rows_v, acc2_o.at[kdi_v.at[i]], add=True)
        return carry

    lax.fori_loop(0, 8, stepB, 0)
    plsc.subcore_barrier()

    pltpu.sync_copy(acc2_o.at[pl.ds(r0, R)], work_v)

    def row_r1(r, carry):
        cv = jnp.maximum(work_v[r, pl.ds(B, 16)], 1.0)
        w0 = par_v[r, pl.ds(0, 16)]
        n0 = par_v[r, pl.ds(16, 16)]
        bb = par_v[r, pl.ds(32, 16)]

        def chunk(j, c2):
            agg = work_v[r, pl.ds(j * 16, 16)] / cv
            x = keA_v[r, pl.ds(j * 16, 16)] * w0 + agg * n0 + bb
            work_v[r, pl.ds(j * 16, 16)] = jnp.maximum(x, 0.0)
            return c2

        lax.fori_loop(0, NJ, chunk, 0)
        keA_v[r, pl.ds(B, 16)] = cv
        for j in range(NJ, BA // 16):
            work_v[r, pl.ds(j * 16, 16)] = jnp.ones((16,), jnp.float32)
        return carry

    lax.fori_loop(0, R, row_r1, 0)
    pltpu.sync_copy(work_v, ke1_o.at[pl.ds(r0, R)])
    plsc.subcore_barrier()


    def stepC(i, carry):
        pltpu.async_copy(ke1_o.at[ksi_v.at[i]], rows_v, sem).wait()
        pltpu.sync_copy(rows_v, acc3_o.at[kdi_v.at[i]], add=True)
        return carry

    lax.fori_loop(0, 8, stepC, 0)
    plsc.subcore_barrier()

    pltpu.sync_copy(acc3_o.at[pl.ds(r0, R)], rows_v.at[pl.ds(0, R)])

    def row_r2(r, carry):
        cv = jnp.maximum(keA_v[r, pl.ds(B, 16)], 1.0)
        w1 = par_v[r, pl.ds(48, 16)]
        n1 = par_v[r, pl.ds(64, 16)]
        bb = par_v[r, pl.ds(80, 16)]

        def chunk(j, c2):
            agg = rows_v[r, pl.ds(j * 16, 16)] / cv
            x = work_v[r, pl.ds(j * 16, 16)] * w1 + agg * n1 + bb
            keA_v[r, pl.ds(j * 16, 16)] = jnp.maximum(x, 0.0)
            return c2

        lax.fori_loop(0, NJ, chunk, 0)
        return carry

    lax.fori_loop(0, R, row_r2, 0)
    pltpu.sync_copy(keA_v, acc1_o.at[pl.ds(r0, R)])
    plsc.subcore_barrier()

    TR = NT // NS
    pltpu.async_copy(acc1_o.at[tis_v], rows_v.at[pl.ds(0, TR)], sem).wait()
    pltpu.sync_copy(rows_v.at[pl.ds(0, TR)], bio_o.at[pl.ds(s * TR, TR)])


def _sc_ke(goT, gks, gkd, kks, kkd, tis, par, zacc):
    mesh = plsc.VectorSubcoreMesh(core_axis_name="c", subcore_axis_name="s",
                                  num_cores=1, num_subcores=NS)
    R = NKE // NS
    f = pl.kernel(
        _sc_ke_body,
        out_type=(jax.ShapeDtypeStruct((NT, BA), jnp.float32),
                  jax.ShapeDtypeStruct((NKE, BA), jnp.float32),
                  jax.ShapeDtypeStruct((NKE, BA), jnp.float32),
                  jax.ShapeDtypeStruct((NKE, BA), jnp.float32),
                  jax.ShapeDtypeStruct((NKE, BA), jnp.float32),
                  jax.ShapeDtypeStruct((NKE, BA), jnp.float32)),
        mesh=mesh,
        scratch_types=[
            pltpu.VMEM((16, CH), jnp.int32),
            pltpu.VMEM((16, CH), jnp.int32),
            pltpu.VMEM((8, CH), jnp.int32),
            pltpu.VMEM((8, CH), jnp.int32),
            pltpu.VMEM((NT // NS,), jnp.int32),
            pltpu.VMEM((CH, BA), jnp.float32),
            pltpu.VMEM((R, BA), jnp.float32),
            pltpu.VMEM((R, BA), jnp.float32),
            pltpu.VMEM((R, 128), jnp.float32),
            pltpu.SemaphoreType.DMA,
        ],
    )
    return f(goT, gks, gkd, kks, kkd, tis, par, zacc)


def _tc_final_body(bio_ref, kw1_ref, kb1_ref, kw2_ref, kb2_ref,
                   bw1_ref, bb1_ref, bw2_ref, bb2_ref,
                   xd_ref, dw1_ref, db1_ref, dw2_ref, db2_ref,
                   pwt_ref, pb_ref, o_ref):
    br = bio_ref[...][:, :B]
    acc = jnp.zeros_like(br)
    for i in range(H):
        acc += jnp.tanh(br * kw1_ref[0, i] + kb1_ref[0, i]) * kw2_ref[0, i]
    br = acc + kb2_ref[0, 0]
    h1 = lax.dot_general(br, bw1_ref[...], (((0,), (0,)), ((), ())),
                         preferred_element_type=jnp.float32)
    h1 = jnp.maximum(h1 + bb1_ref[...], 0.0)
    h2 = jnp.dot(h1, bw2_ref[...], preferred_element_type=jnp.float32)
    h2 = jnp.maximum(h2 + bb2_ref[...], 0.0)
    d1 = jnp.dot(xd_ref[...], dw1_ref[...], preferred_element_type=jnp.float32)
    d1 = jnp.maximum(d1 + db1_ref[...], 0.0)
    d2 = jnp.dot(d1, dw2_ref[...], preferred_element_type=jnp.float32)
    d2 = jnp.maximum(d2 + db2_ref[...], 0.0)
    pw = pwt_ref[...]
    res = (jnp.sum(h2 * pw[:, :128], axis=1, keepdims=True)
           + jnp.sum(d2 * pw[:, 128:], axis=1, keepdims=True)
           + pb_ref[0, 0])
    o_ref[...] = res


def _tc_final(bio_raw, kw1, kb1, kw2, kb2, bw1, bb1, bw2, bb2,
              xd, dw1, db1, dw2, db2, pwt, pb):
    return pl.pallas_call(
        _tc_final_body,
        out_shape=jax.ShapeDtypeStruct((B, 1), jnp.float32),
    )(bio_raw, kw1, kb1, kw2, kb2, bw1, bb1, bw2, bb2,
      xd, dw1, db1, dw2, db2, pwt, pb)


def kernel(input_tensor, gene_W1, gene_b1, gene_W2, gene_b2, go_enc_W, go_enc_b,
           go_dec_W1, go_dec_b1, go_dec_W2, go_dec_b2,
           ke_ws0, ke_wn0, ke_b0, ke_ws1, ke_wn1, ke_b1,
           kel_W1, kel_b1, kel_W2, kel_b2,
           bio_W1, bio_b1, bio_W2, bio_b2,
           drug_W1, drug_b1, drug_W2, drug_b2,
           pred_W, pred_b,
           gene_go, go_ke, ke_ke, tissue):
    r8 = lambda v: v.reshape(1, H)
    r1 = lambda v: v.reshape(1, 1)
    b16 = lambda v: jnp.broadcast_to(v[:, None], (v.shape[0], 16))

    xT = input_tensor[:, :NG].T
    geneT = _tc_gene(xT, r8(gene_W1), r8(gene_b1), r8(gene_W2), r1(gene_b2))

    zacc = jnp.zeros((CH, BA), jnp.float32)

    gg_src = gene_go[0].reshape(NC * NS, 16, CH)
    gg_dst = gene_go[1].reshape(NC * NS, 16, CH)
    gsum = _sc_gene(geneT, gg_src, gg_dst, zacc)

    goT = _tc_go(gsum, go_enc_W, b16(go_enc_b),
                 r8(go_dec_W1), r8(go_dec_b1), r8(go_dec_W2), r1(go_dec_b2))

    gk_src = go_ke[0].reshape(NS, 16, CH)
    gk_dst = go_ke[1].reshape(NS, 16, CH)
    kk_src = ke_ke[0].reshape(NS, 8, CH)
    kk_dst = ke_ke[1].reshape(NS, 8, CH)
    tis = tissue.reshape(NS, NT // NS)
    par = jnp.concatenate(
        [b16(ke_ws0), b16(ke_wn0), b16(ke_b0),
         b16(ke_ws1), b16(ke_wn1), b16(ke_b1),
         jnp.zeros((NKE, 32), jnp.float32)], axis=1)
    outs = _sc_ke(goT, gk_src, gk_dst, kk_src, kk_dst, tis, par, zacc)
    bio_raw = outs[0]

    return _tc_final(bio_raw, r8(kel_W1), r8(kel_b1), r8(kel_W2), r1(kel_b2),
                     bio_W1, bio_b1.reshape(1, -1), bio_W2, bio_b2.reshape(1, -1),
                     input_tensor[:, NG:], drug_W1, drug_b1.reshape(1, -1),
                     drug_W2, drug_b2.reshape(1, -1),
                     pred_W.T, pred_b.reshape(1, 1))
